# baseline (speedup 1.0000x reference)
import functools

import jax
import jax.numpy as jnp
import numpy as np
from jax import lax
from jax.experimental import pallas as pl
from jax.experimental.pallas import tpu as pltpu
from jax.experimental.pallas import tpu_sc as plsc

B = 16384
H = 50
D = 32
HD = D // 2
OUT_D = 3 * D
NC = 2
NS = 16
NW = NC * NS
RW = B // NW
C = 32
G = RW // C
CH = C * H
CHP = 1664
L = 16


def _sc_kernel(user_hbm, mem_hbm, utab_hbm, itabl_hbm, itabr_hbm,
               rtab_hbm, btab_hbm, out_hbm,
               idx2_a, idx_a, rowsl_a, rowsr_a,
               idx2_b, idx_b, rowsl_b, rowsr_b,
               out_v, uidx_v, user_v, rtab_v, btab_v,
               sem_al, sem_ar, sem_bl, sem_br, sem_u):
    sid = lax.axis_index("s")
    wid = sid * NC + lax.axis_index("c")
    base = wid * RW

    inv_h = jnp.float32(1.0 / H)

    pltpu.sync_copy(rtab_hbm, rtab_v)
    pltpu.sync_copy(btab_hbm, btab_v)

    def prefetch(g, idx2_v, idx_v, rowsl_v, rowsr_v, seml, semr):
        r0 = base + g * C
        pltpu.sync_copy(mem_hbm.at[:, pl.ds(r0, C)], idx2_v)

        def flat_body(i, _):
            sl = pl.ds(i * L, L)
            v = plsc.load_gather(idx2_v, [rtab_v[sl], btab_v[sl]])
            idx_v[sl] = v.astype(jnp.int32)
            return 0
        lax.fori_loop(0, CHP // L, flat_body, 0)
        pltpu.async_copy(itabl_hbm.at[idx_v], rowsl_v, seml)
        pltpu.async_copy(itabr_hbm.at[idx_v], rowsr_v, semr)

    def finish(g, idx_v, rowsl_v, rowsr_v, seml, semr):
        r0 = base + g * C
        pltpu.make_async_copy(itabl_hbm.at[idx_v], rowsl_v, seml).wait()
        pltpu.make_async_copy(itabr_hbm.at[idx_v], rowsr_v, semr).wait()
        pltpu.sync_copy(user_hbm.at[pl.ds(r0, C)], uidx_v)
        pltpu.async_copy(utab_hbm.at[uidx_v], user_v, sem_u).wait()

        def row_body(r, _):
            acc0 = jnp.zeros((L,), jnp.float32)
            acc1 = jnp.zeros((L,), jnp.float32)
            for h in range(H):
                acc0 = acc0 + rowsl_v[h * C + r, pl.ds(0, L)]
                acc1 = acc1 + rowsr_v[h * C + r, pl.ds(0, L)]
            m0 = acc0 * inv_h
            m1 = acc1 * inv_h
            u0 = user_v[r, pl.ds(0, L)]
            u1 = user_v[r, pl.ds(L, L)]
            out_v[r, pl.ds(0, L)] = m0
            out_v[r, pl.ds(L, L)] = m1
            out_v[r, pl.ds(D, L)] = m0 * u0
            out_v[r, pl.ds(D + L, L)] = m1 * u1
            out_v[r, pl.ds(2 * D, L)] = u0
            out_v[r, pl.ds(2 * D + L, L)] = u1
            return 0
        lax.fori_loop(0, C, row_body, 0)
        pltpu.sync_copy(out_v, out_hbm.at[pl.ds(r0, C)])

    prefetch(0, idx2_a, idx_a, rowsl_a, rowsr_a, sem_al, sem_ar)

    def pair_body(k, _):
        g0 = k * 2

        @pl.when(g0 + 1 < G)
        def _():
            prefetch(g0 + 1, idx2_b, idx_b, rowsl_b, rowsr_b, sem_bl, sem_br)
        finish(g0, idx_a, rowsl_a, rowsr_a, sem_al, sem_ar)

        @pl.when(g0 + 2 < G)
        def _():
            prefetch(g0 + 2, idx2_a, idx_a, rowsl_a, rowsr_a, sem_al, sem_ar)

        @pl.when(g0 + 1 < G)
        def _():
            finish(g0 + 1, idx_b, rowsl_b, rowsr_b, sem_bl, sem_br)
        return 0

    lax.fori_loop(0, (G + 1) // 2, pair_body, 0)


@jax.jit
def _run(user, memory, user_table, itab_l, itab_r, r_tab, b_tab):
    mesh = plsc.VectorSubcoreMesh(core_axis_name="c", subcore_axis_name="s")
    f = functools.partial(
        pl.kernel,
        mesh=mesh,
        compiler_params=pltpu.CompilerParams(use_tc_tiling_on_sc=False,
                                             needs_layout_passes=False),
        out_type=jax.ShapeDtypeStruct((B, OUT_D), jnp.float32),
        scratch_types=[
            pltpu.VMEM((H, C), jnp.float32),
            pltpu.VMEM((CHP,), jnp.int32),
            pltpu.VMEM((CHP, HD), jnp.float32),
            pltpu.VMEM((CHP, HD), jnp.float32),
            pltpu.VMEM((H, C), jnp.float32),
            pltpu.VMEM((CHP,), jnp.int32),
            pltpu.VMEM((CHP, HD), jnp.float32),
            pltpu.VMEM((CHP, HD), jnp.float32),
            pltpu.VMEM((C, OUT_D), jnp.float32),
            pltpu.VMEM((C,), jnp.int32),
            pltpu.VMEM((C, D), jnp.float32),
            pltpu.VMEM((CHP,), jnp.int32),
            pltpu.VMEM((CHP,), jnp.int32),
            pltpu.SemaphoreType.DMA,
            pltpu.SemaphoreType.DMA,
            pltpu.SemaphoreType.DMA,
            pltpu.SemaphoreType.DMA,
            pltpu.SemaphoreType.DMA,
        ],
    )(_sc_kernel)
    return f(user, memory, user_table, itab_l, itab_r, r_tab, b_tab)


_P = np.arange(CHP)
_R_TAB = np.where(_P < CH, _P // C, 0).astype(np.int32)
_B_TAB = np.where(_P < CH, _P % C, 0).astype(np.int32)


def kernel(user, memory, user_table, item_table):
    return _run(user, memory.astype(jnp.float32).T, user_table,
                item_table[:, :HD], item_table[:, HD:], _R_TAB, _B_TAB)


# speedup vs baseline: 2.1624x; 2.1624x over previous
import functools

import jax
import jax.numpy as jnp
import numpy as np
from jax import lax
from jax.experimental import pallas as pl
from jax.experimental.pallas import tpu as pltpu
from jax.experimental.pallas import tpu_sc as plsc

B = 16384
H = 50
D = 32
OUT_D = 3 * D
NC = 2
NS = 16
NW = NC * NS
RW = B // NW
C = 32
G = RW // C
CH = C * H
CHP = 1664
L = 16


def _sc_kernel(user_hbm, mem_hbm, utab_hbm, itab_hbm, rtab_hbm, btab_hbm,
               out_hbm,
               idx2_a, idx_a, rows_a, idx2_b, idx_b, rows_b,
               out_v, uidx_v, user_v, rtab_v, btab_v,
               sem_a, sem_b, sem_u):
    sid = lax.axis_index("s")
    wid = sid * NC + lax.axis_index("c")
    base = wid * RW

    inv_h = jnp.float32(1.0 / H)

    pltpu.sync_copy(rtab_hbm, rtab_v)
    pltpu.sync_copy(btab_hbm, btab_v)

    def prefetch(g, idx2_v, idx_v, rows_v, sem):
        r0 = base + g * C
        pltpu.sync_copy(mem_hbm.at[:, pl.ds(r0, C)], idx2_v)

        def flat_body(i, _):
            sl = pl.ds(i * L, L)
            v = plsc.load_gather(idx2_v, [rtab_v[sl], btab_v[sl]])
            idx_v[sl] = v.astype(jnp.int32)
            return 0
        lax.fori_loop(0, CHP // L, flat_body, 0)
        pltpu.async_copy(itab_hbm.at[idx_v], rows_v, sem)

    def finish(g, idx_v, rows_v, sem):
        r0 = base + g * C
        pltpu.make_async_copy(itab_hbm.at[idx_v], rows_v, sem).wait()
        pltpu.sync_copy(user_hbm.at[pl.ds(r0, C)], uidx_v)
        pltpu.async_copy(utab_hbm.at[uidx_v], user_v, sem_u).wait()

        def row_body(r, _):
            acc0 = jnp.zeros((L,), jnp.float32)
            acc1 = jnp.zeros((L,), jnp.float32)
            for h in range(H):
                acc0 = acc0 + rows_v[h * C + r, pl.ds(0, L)]
                acc1 = acc1 + rows_v[h * C + r, pl.ds(L, L)]
            m0 = acc0 * inv_h
            m1 = acc1 * inv_h
            u0 = user_v[r, pl.ds(0, L)]
            u1 = user_v[r, pl.ds(L, L)]
            out_v[r, pl.ds(0, L)] = m0
            out_v[r, pl.ds(L, L)] = m1
            out_v[r, pl.ds(D, L)] = m0 * u0
            out_v[r, pl.ds(D + L, L)] = m1 * u1
            out_v[r, pl.ds(2 * D, L)] = u0
            out_v[r, pl.ds(2 * D + L, L)] = u1
            return 0
        lax.fori_loop(0, C, row_body, 0)
        pltpu.sync_copy(out_v, out_hbm.at[pl.ds(r0, C)])

    prefetch(0, idx2_a, idx_a, rows_a, sem_a)

    def pair_body(k, _):
        g0 = k * 2

        @pl.when(g0 + 1 < G)
        def _():
            prefetch(g0 + 1, idx2_b, idx_b, rows_b, sem_b)
        finish(g0, idx_a, rows_a, sem_a)

        @pl.when(g0 + 2 < G)
        def _():
            prefetch(g0 + 2, idx2_a, idx_a, rows_a, sem_a)

        @pl.when(g0 + 1 < G)
        def _():
            finish(g0 + 1, idx_b, rows_b, sem_b)
        return 0

    lax.fori_loop(0, (G + 1) // 2, pair_body, 0)


@jax.jit
def _run(user, memory, user_table, item_table, r_tab, b_tab):
    mesh = plsc.VectorSubcoreMesh(core_axis_name="c", subcore_axis_name="s")
    f = functools.partial(
        pl.kernel,
        mesh=mesh,
        compiler_params=pltpu.CompilerParams(use_tc_tiling_on_sc=False,
                                             needs_layout_passes=False),
        out_type=jax.ShapeDtypeStruct((B, OUT_D), jnp.float32),
        scratch_types=[
            pltpu.VMEM((H, C), jnp.float32),
            pltpu.VMEM((CHP,), jnp.int32),
            pltpu.VMEM((CHP, D), jnp.float32),
            pltpu.VMEM((H, C), jnp.float32),
            pltpu.VMEM((CHP,), jnp.int32),
            pltpu.VMEM((CHP, D), jnp.float32),
            pltpu.VMEM((C, OUT_D), jnp.float32),
            pltpu.VMEM((C,), jnp.int32),
            pltpu.VMEM((C, D), jnp.float32),
            pltpu.VMEM((CHP,), jnp.int32),
            pltpu.VMEM((CHP,), jnp.int32),
            pltpu.SemaphoreType.DMA,
            pltpu.SemaphoreType.DMA,
            pltpu.SemaphoreType.DMA,
        ],
    )(_sc_kernel)
    return f(user, memory, user_table, item_table, r_tab, b_tab)


_P = np.arange(CHP)
_R_TAB = np.where(_P < CH, _P // C, 0).astype(np.int32)
_B_TAB = np.where(_P < CH, _P % C, 0).astype(np.int32)


def kernel(user, memory, user_table, item_table):
    return _run(user, memory.astype(jnp.float32).T, user_table, item_table,
                _R_TAB, _B_TAB)
